# baseline (speedup 1.0000x reference)
import functools

import jax
import jax.numpy as jnp
import numpy as np
from jax import lax
from jax.experimental import pallas as pl
from jax.experimental.pallas import tpu as pltpu
from jax.experimental.pallas import tpu_sc as plsc

_VOCAB = 1000000
_D = 32
_B = 16384
_H = 20
_L = 16
_NC, _NS = 2, 16
_NW = _NC * _NS
_BAGS_PER_W = _B // _NW
_ROWS_PER_W = _BAGS_PER_W * _H
_CHUNK = 32
_NCHUNK = _BAGS_PER_W // _CHUNK
_ROWS_PER_CHUNK = _CHUNK * _H
_IDX_COLS = 128
_GATHERS_PER_CHUNK = _ROWS_PER_CHUNK // _IDX_COLS
_IDX_ROWS_PER_W = _ROWS_PER_W // _IDX_COLS
_OUT_ROWS_PER_CHUNK = _CHUNK * _D // 128
_OUT_ROWS_PER_W = _BAGS_PER_W * _D // 128


def _quant_mag(z):
    return jnp.where(z > 5.0, 6.0,
           jnp.where(z > 3.5, 4.0,
           jnp.where(z > 2.5, 3.0,
           jnp.where(z > 1.75, 2.0,
           jnp.where(z > 1.25, 1.5,
           jnp.where(z > 0.75, 1.0,
           jnp.where(z > 0.25, 0.5, 0.0)))))))


def _idx_body(x_ref, out_ref):
    xf = x_ref[...].astype(jnp.float32)
    col = lax.broadcasted_iota(jnp.int32, xf.shape, 1)
    is0 = col < _L
    ax = jnp.abs(xf)
    m0 = jnp.max(jnp.where(is0, ax, 0.0), axis=1, keepdims=True)
    m1 = jnp.max(jnp.where(is0, 0.0, ax), axis=1, keepdims=True)
    amax = jnp.where(is0, m0, m1)
    scale = jnp.where(amax > 0, amax / 6.0, 1.0)
    y = xf / scale
    z = jnp.abs(y)
    qm = _quant_mag(z)
    dq = jnp.where(y < 0, -qm, qm) * scale
    r = lax.round(dq, lax.RoundingMethod.TO_NEAREST_EVEN)
    out_ref[...] = jnp.clip(r, 0.0, float(_VOCAB - 1)).astype(jnp.int32)


_idx_call = pl.pallas_call(
    _idx_body,
    out_shape=jax.ShapeDtypeStruct((_B, _H), jnp.int32),
    grid=(16,),
    in_specs=[pl.BlockSpec((_B // 16, _H), lambda i: (i, 0))],
    out_specs=pl.BlockSpec((_B // 16, _H), lambda i: (i, 0)),
)


_GATHER_DNUMS = lax.GatherDimensionNumbers(
    offset_dims=(), collapsed_slice_dims=(0,), start_index_map=(0,))


def _shuffle(v, perm):
    return lax.gather(v, perm[:, None], _GATHER_DNUMS, (1,),
                      mode=lax.GatherScatterMode.PROMISE_IN_BOUNDS)


def _lanemax(v):
    for s in (8, 4, 2, 1):
        perm = lax.iota(jnp.int32, _L) ^ s
        v = jnp.maximum(v, _shuffle(v, perm))
    return v


_THR = tuple(np.float32(v) for v in
             (5.0 / 6, 7.0 / 12, 5.0 / 12, 7.0 / 24, 5.0 / 24, 1.0 / 8,
              1.0 / 24))
_QC = tuple(np.float32(v) for v in
            (1.0, 2.0 / 3, 1.0 / 2, 1.0 / 3, 1.0 / 4, 1.0 / 6, 1.0 / 12))


def _qdq16(w):
    u = jnp.abs(w)
    a = _lanemax(u)
    qc = jnp.where(u > a * _THR[0], _QC[0],
         jnp.where(u > a * _THR[1], _QC[1],
         jnp.where(u > a * _THR[2], _QC[2],
         jnp.where(u > a * _THR[3], _QC[3],
         jnp.where(u > a * _THR[4], _QC[4],
         jnp.where(u > a * _THR[5], _QC[5],
         jnp.where(u > a * _THR[6], _QC[6], np.float32(0.0))))))))
    d = qc * a
    return jnp.where(w < 0, -d, d)


_LANE = None


def _sc_body(idx_hbm, table_hbm, out_hbm, stage_v, sf_v, pidx_v, rows_v,
             out_v, sem):
    wid = lax.axis_index("s") * _NC + lax.axis_index("c")
    pltpu.sync_copy(idx_hbm.at[pl.ds(wid * _ROWS_PER_W, _ROWS_PER_W)],
                    stage_v)

    def prep_body(r, carry):
        for c8 in range(_IDX_COLS // _L):
            off = r * _IDX_COLS + c8 * _L
            v = stage_v[pl.ds(off, _L)]
            pidx_v[r, pl.ds(c8 * _L, _L)] = v >> 2
            sf_v[pl.ds(off, _L)] = (v & 3).astype(jnp.float32)
        return carry

    lax.fori_loop(0, _IDX_ROWS_PER_W, prep_body, 0)

    def chunk_body(c, carry):
        copies = []
        for g in range(_GATHERS_PER_CHUNK):
            copies.append(pltpu.async_copy(
                table_hbm.at[pidx_v.at[c * _GATHERS_PER_CHUNK + g]],
                rows_v.at[pl.ds(g * _IDX_COLS, _IDX_COLS)], sem))
        for cp in copies:
            cp.wait()

        def bag_body(b, carry2):
            e0 = c * _ROWS_PER_CHUNK + b * _H
            sfa = sf_v[pl.ds(e0, _L)]
            sfb = sf_v[pl.ds(e0 + _L, _L)]
            rbase = b * _H
            acc0 = jnp.zeros((_L,), jnp.float32)
            acc1 = jnp.zeros((_L,), jnp.float32)
            for k in range(_H):
                src, kk = (sfa, k) if k < _L else (sfb, k - _L)
                sfk = _shuffle(src, jnp.full((_L,), kk, jnp.int32))
                hi = (jnp.maximum(sfk - 1.0, 0.0)
                      - jnp.maximum(sfk - 2.0, 0.0))
                od = sfk - 2.0 * hi
                r = rbase + k
                v = [rows_v[r, pl.ds(h * _L, _L)] for h in range(8)]
                lo0 = v[0] + od * (v[2] - v[0])
                hi0 = v[4] + od * (v[6] - v[4])
                w0 = lo0 + hi * (hi0 - lo0)
                lo1 = v[1] + od * (v[3] - v[1])
                hi1 = v[5] + od * (v[7] - v[5])
                w1 = lo1 + hi * (hi1 - lo1)
                acc0 = acc0 + _qdq16(w0)
                acc1 = acc1 + _qdq16(w1)
            orow = b >> 2
            ocol = (b & 3) * _D
            out_v[orow, pl.ds(ocol, _L)] = acc0 / float(_H)
            out_v[orow, pl.ds(ocol + _L, _L)] = acc1 / float(_H)
            return carry2

        lax.fori_loop(0, _CHUNK, bag_body, 0)
        pltpu.sync_copy(
            out_v,
            out_hbm.at[pl.ds(wid * _OUT_ROWS_PER_W + c * _OUT_ROWS_PER_CHUNK,
                             _OUT_ROWS_PER_CHUNK)])
        return carry

    lax.fori_loop(0, _NCHUNK, chunk_body, 0)


@functools.cache
def _sc_call():
    return pl.kernel(
        _sc_body,
        out_type=jax.ShapeDtypeStruct((_B * _D // 128, 128), jnp.float32),
        mesh=plsc.VectorSubcoreMesh(core_axis_name="c", subcore_axis_name="s"),
        compiler_params=pltpu.CompilerParams(use_tc_tiling_on_sc=False),
        scratch_types=[
            pltpu.VMEM((_ROWS_PER_W,), jnp.int32),
            pltpu.VMEM((_ROWS_PER_W + 2 * _L,), jnp.float32),
            pltpu.VMEM((_IDX_ROWS_PER_W, _IDX_COLS), jnp.int32),
            pltpu.VMEM((_ROWS_PER_CHUNK, 128), jnp.float32),
            pltpu.VMEM((_OUT_ROWS_PER_CHUNK, 128), jnp.float32),
            pltpu.SemaphoreType.DMA,
        ],
    )


def kernel(x, weight):
    x = x.astype(jnp.int32)
    idx = _idx_call(x)
    idx_flat = idx.reshape(_B * _H)
    w128 = weight.reshape(_VOCAB // 4, 128)
    out = _sc_call()(idx_flat, w128)
    return out.reshape(_B, _D)
